# baseline (speedup 1.0000x reference)
import jax
import jax.numpy as jnp
from jax import lax
from jax.experimental import pallas as pl
from jax.experimental.pallas import tpu as pltpu
from jax.experimental.pallas import tpu_sc as plsc

_VOCAB = 16
_HIDDEN = 128
_BLK = 16384

_NW = 32
_K = 4
_ROWS_PER_W = 200
_NCHUNK = _ROWS_PER_W // _K


def _tc_kernel(ids_ref, emb_ref, out_ref):
    ids = ids_ref[...]
    iota = lax.broadcasted_iota(jnp.int32, (1, _VOCAB), 1)
    one_hot = (ids == iota).astype(jnp.float32)
    out_ref[...] = lax.dot_general(
        one_hot, emb_ref[...],
        (((1,), (0,)), ((), ())),
        preferred_element_type=jnp.float32,
    )


def _tc_lookup(ids_col, embedding, n):
    return pl.pallas_call(
        _tc_kernel,
        grid=(n // _BLK,),
        in_specs=[
            pl.BlockSpec((_BLK, 1), lambda i: (i, 0)),
            pl.BlockSpec((_VOCAB, _HIDDEN), lambda i: (0, 0)),
        ],
        out_specs=pl.BlockSpec((_BLK, _HIDDEN), lambda i: (i, 0)),
        out_shape=jax.ShapeDtypeStruct((n, _HIDDEN), jnp.float32),
    )(ids_col, embedding)


def _sc_body(ids_hbm, emb_hbm, out_hbm, idx_v, rows_v, sem):
    c = lax.axis_index("c")
    s = lax.axis_index("s")
    wid = s * 2 + c
    row0 = wid * _ROWS_PER_W
    pltpu.sync_copy(ids_hbm.at[pl.ds(row0, _ROWS_PER_W)], idx_v)

    def body(i, carry):
        r = i * _K
        copies = [
            pltpu.async_copy(
                emb_hbm.at[idx_v.at[r + j]],
                rows_v.at[pl.ds(j * 128, 128)],
                sem,
            )
            for j in range(_K)
        ]
        for cp in copies:
            cp.wait()
        pltpu.sync_copy(
            rows_v, out_hbm.at[pl.ds((row0 + r) * 128, _K * 128)])
        return carry

    lax.fori_loop(0, _NCHUNK, body, 0)


def _sc_lookup(ids_2d, embedding, n):
    mesh = plsc.VectorSubcoreMesh(core_axis_name="c", subcore_axis_name="s")
    k = pl.kernel(
        _sc_body,
        mesh=mesh,
        out_type=jax.ShapeDtypeStruct((n, _HIDDEN), jnp.float32),
        scratch_types=[
            pltpu.VMEM((_ROWS_PER_W, 128), jnp.int32),
            pltpu.VMEM((_K * 128, _HIDDEN), jnp.float32),
            pltpu.SemaphoreType.DMA,
        ],
    )
    return k(ids_2d, embedding)


def kernel(input_ids, embedding):
    batch, seq = input_ids.shape
    n = batch * seq
    ids_flat = input_ids.reshape(n).astype(jnp.int32)
    out0 = _tc_lookup(ids_flat.reshape(n, 1), embedding, n)
    out1 = _sc_lookup(ids_flat.reshape(n // 128, 128), embedding, n)
    return (out0.reshape(batch, seq, _HIDDEN),
            out1.reshape(batch, seq, _HIDDEN))


# speedup vs baseline: 5.5691x; 5.5691x over previous
import jax
import jax.numpy as jnp
from jax import lax
from jax.experimental import pallas as pl
from jax.experimental.pallas import tpu as pltpu
from jax.experimental.pallas import tpu_sc as plsc

_VOCAB = 16
_HIDDEN = 128
_BLK = 16384

_NW = 32
_K = 2
_ROWS_PER_W = 200
_NCHUNK = _ROWS_PER_W // _K


def _tc_kernel(ids_ref, emb_ref, out_ref):
    ids = ids_ref[...]
    iota = lax.broadcasted_iota(jnp.int32, (1, _VOCAB), 1)
    one_hot = (ids == iota).astype(jnp.float32)
    out_ref[...] = lax.dot_general(
        one_hot, emb_ref[...],
        (((1,), (0,)), ((), ())),
        preferred_element_type=jnp.float32,
    )


def _tc_lookup(ids_col, embedding, n):
    return pl.pallas_call(
        _tc_kernel,
        grid=(n // _BLK,),
        in_specs=[
            pl.BlockSpec((_BLK, 1), lambda i: (i, 0)),
            pl.BlockSpec((_VOCAB, _HIDDEN), lambda i: (0, 0)),
        ],
        out_specs=pl.BlockSpec((_BLK, _HIDDEN), lambda i: (i, 0)),
        out_shape=jax.ShapeDtypeStruct((n, _HIDDEN), jnp.float32),
    )(ids_col, embedding)


def _sc_body(ids_hbm, emb_hbm, out_hbm, idx_v, table_v,
             rows0_v, rows1_v, gsem0, gsem1):
    c = lax.axis_index("c")
    s = lax.axis_index("s")
    wid = s * 2 + c
    row0 = wid * _ROWS_PER_W
    pltpu.sync_copy(ids_hbm.at[pl.ds(row0, _ROWS_PER_W)], idx_v)

    @pl.when(s == 0)
    def _():
        pltpu.sync_copy(emb_hbm, table_v)

    plsc.subcore_barrier()

    def gather(r, buf, sem):
        cps = [
            pltpu.async_copy(
                table_v.at[idx_v.at[r + j]],
                buf.at[pl.ds(j * 128, 128)],
                sem,
            )
            for j in range(_K)
        ]
        return cps

    def body(i, carry):
        ra = 2 * i * _K
        rb = ra + _K
        cps_a = gather(ra, rows0_v, gsem0)
        cps_b = gather(rb, rows1_v, gsem1)
        for cp in cps_a:
            cp.wait()
        pltpu.sync_copy(
            rows0_v, out_hbm.at[pl.ds((row0 + ra) * 128, _K * 128)])
        for cp in cps_b:
            cp.wait()
        pltpu.sync_copy(
            rows1_v, out_hbm.at[pl.ds((row0 + rb) * 128, _K * 128)])
        return carry

    lax.fori_loop(0, _NCHUNK // 2, body, 0)


def _sc_lookup(ids_2d, embedding, n):
    mesh = plsc.VectorSubcoreMesh(core_axis_name="c", subcore_axis_name="s")
    k = pl.kernel(
        _sc_body,
        mesh=mesh,
        out_type=jax.ShapeDtypeStruct((n, _HIDDEN), jnp.float32),
        scratch_types=[
            pltpu.VMEM((_ROWS_PER_W, 128), jnp.int32),
            pltpu.VMEM_SHARED((_VOCAB, _HIDDEN), jnp.float32),
            pltpu.VMEM((_K * 128, _HIDDEN), jnp.float32),
            pltpu.VMEM((_K * 128, _HIDDEN), jnp.float32),
            pltpu.SemaphoreType.DMA,
            pltpu.SemaphoreType.DMA,
        ],
    )
    return k(ids_2d, embedding)


def kernel(input_ids, embedding):
    batch, seq = input_ids.shape
    n = batch * seq
    ids_flat = input_ids.reshape(n).astype(jnp.int32)
    out0 = _tc_lookup(ids_flat.reshape(n, 1), embedding, n)
    out1 = _sc_lookup(ids_flat.reshape(n // 128, 128), embedding, n)
    return (out0.reshape(batch, seq, _HIDDEN),
            out1.reshape(batch, seq, _HIDDEN))
